# baseline (speedup 1.0000x reference)
import functools

import jax
import jax.numpy as jnp
from jax import lax
from jax.experimental import pallas as pl
from jax.experimental.pallas import tpu as pltpu
from jax.experimental.pallas import tpu_sc as plsc

N = 10000
E = 320000
D = 128
DH = D // 2

NC = 2
NS = 16
EPT = E // NS
K = 128
NCHUNK = EPT // K
KTAIL = EPT - NCHUNK * K
NBUF = 4
ROWS_PER_TILE = 640
NPAD = ROWS_PER_TILE * NS
VL = 16


def _sc_segment_sum(z2, edge_index):
  mesh = plsc.VectorSubcoreMesh(core_axis_name="c", subcore_axis_name="s")

  @functools.partial(
      pl.kernel,
      mesh=mesh,
      compiler_params=pltpu.CompilerParams(use_tc_tiling_on_sc=False),
      out_type=jax.ShapeDtypeStruct((NPAD, D), jnp.float32),
      scratch_types=[
          pltpu.VMEM((EPT + 3 * K - EPT % K,), jnp.int32),
          pltpu.VMEM((EPT,), jnp.int32),
          [pltpu.VMEM((K, DH), jnp.float32)] * NBUF,
          pltpu.VMEM_SHARED((NPAD, DH), jnp.float32),
          [pltpu.SemaphoreType.DMA] * NBUF,
          [pltpu.SemaphoreType.DMA] * NBUF,
      ],
  )
  def k(z2_hbm, ei_hbm, out_hbm, src_v, dst_v, rows, acc, gsem, ssem):
    c = lax.axis_index("c")
    s = lax.axis_index("s")

    def zbody(i, carry):
      for u in range(DH // VL):
        rows[0][i, pl.ds(u * VL, VL)] = jnp.zeros((VL,), jnp.float32)
      return carry

    lax.fori_loop(0, K, zbody, 0)
    r0 = s * ROWS_PER_TILE
    for q in range(ROWS_PER_TILE // K):
      pltpu.sync_copy(rows[0],
                      acc.at[pl.ds(r0 + q * K, K)])
    pltpu.sync_copy(ei_hbm.at[0, pl.ds(s * EPT, EPT)],
                    src_v.at[pl.ds(0, EPT)])
    pltpu.sync_copy(ei_hbm.at[1, pl.ds(s * EPT, EPT)], dst_v)

    def transform(j):
      for u in range(K // VL):
        off = j * K + u * VL
        src_v[pl.ds(off, VL)] = src_v[pl.ds(off, VL)] * 2 + c

    transform(0)
    plsc.subcore_barrier()

    def gidx(j):
      return src_v.at[pl.ds(j * K, K)]

    def didx(j):
      return dst_v.at[pl.ds(j * K, K)]

    def wait_gather(j, v):
      pltpu.make_async_copy(z2_hbm.at[gidx(j)], rows[v], gsem[v]).wait()

    def start_scatter(j, v):
      pltpu.async_copy(rows[v], acc.at[didx(j)], ssem[v], add=True)

    def wait_scatter(j, v):
      pltpu.make_async_copy(rows[v], acc.at[didx(j)], ssem[v]).wait()

    def body(i, carry):
      j0 = NBUF * i
      for v in range(NBUF):
        j = j0 + v

        @pl.when(j >= NBUF)
        def _():
          wait_scatter(j - NBUF, v)

        pltpu.async_copy(z2_hbm.at[gidx(j)], rows[v], gsem[v])
        transform(j + 1)

        @pl.when(j >= NBUF - 1)
        def _():
          jl = j - (NBUF - 1)
          wait_gather(jl, (v + 1) % NBUF)
          start_scatter(jl, (v + 1) % NBUF)

      return carry

    lax.fori_loop(0, NCHUNK // NBUF, body, 0)
    wait_scatter(NCHUNK - NBUF, (NCHUNK - NBUF) % NBUF)
    for r in range(NCHUNK - NBUF + 1, NCHUNK):
      v = r % NBUF
      wait_gather(r, v)
      pltpu.sync_copy(rows[v], acc.at[didx(r)], add=True)

    toff = NCHUNK * K
    tsrc = src_v.at[pl.ds(toff, KTAIL)]
    tdst = dst_v.at[pl.ds(toff, KTAIL)]
    trows = rows[0].at[pl.ds(0, KTAIL)]
    pltpu.async_copy(z2_hbm.at[tsrc], trows, gsem[0]).wait()
    pltpu.sync_copy(trows, acc.at[tdst], add=True)

    plsc.subcore_barrier()

    pltpu.sync_copy(acc.at[pl.ds(r0, ROWS_PER_TILE)],
                    out_hbm.at[pl.ds(r0, ROWS_PER_TILE), pl.ds(c * DH, DH)])

  return k(z2, edge_index)


def _tc_zr_body(z_ref, wr_ref, b1_ref, o_ref):
  o_ref[...] = (jnp.dot(z_ref[...].astype(jnp.bfloat16),
                        wr_ref[...].astype(jnp.bfloat16),
                        preferred_element_type=jnp.float32) + b1_ref[...])


def _tc_zr(z, W_root, b1):
  BN = 2000
  return pl.pallas_call(
      _tc_zr_body,
      grid=(N // BN,),
      in_specs=[
          pl.BlockSpec((BN, D), lambda i: (i, 0)),
          pl.BlockSpec((D, D), lambda i: (0, 0)),
          pl.BlockSpec((1, D), lambda i: (0, 0)),
      ],
      out_specs=pl.BlockSpec((BN, D), lambda i: (i, 0)),
      out_shape=jax.ShapeDtypeStruct((N, D), jnp.float32),
  )(z, W_root, b1)


def _tc_body(agg_ref, zr_ref, wn_ref, wo_ref, b2_ref, o_ref):
  h = zr_ref[...] + jnp.dot(agg_ref[...].astype(jnp.bfloat16),
                            wn_ref[...].astype(jnp.bfloat16),
                            preferred_element_type=jnp.float32)
  h = jnp.maximum(h, 0.0)
  o_ref[...] = jnp.dot(h.astype(jnp.bfloat16),
                       wo_ref[...].astype(jnp.bfloat16),
                       preferred_element_type=jnp.float32) + b2_ref[...]


def _tc_decoder(agg, zr, W_nbr, W_out, b2):
  BN = 2000
  return pl.pallas_call(
      _tc_body,
      grid=(N // BN,),
      in_specs=[
          pl.BlockSpec((BN, D), lambda i: (i, 0)),
          pl.BlockSpec((BN, D), lambda i: (i, 0)),
          pl.BlockSpec((D, D), lambda i: (0, 0)),
          pl.BlockSpec((D, D), lambda i: (0, 0)),
          pl.BlockSpec((1, D), lambda i: (0, 0)),
      ],
      out_specs=pl.BlockSpec((BN, D), lambda i: (i, 0)),
      out_shape=jax.ShapeDtypeStruct((N, D), jnp.float32),
  )(agg, zr, W_nbr, W_out, b2)


def kernel(z, edge_index, W_root, W_nbr, b1, W_out, b2):
  z2 = z.reshape(2 * N, DH)
  zr = _tc_zr(z, W_root, b1.reshape(1, D))
  agg = _sc_segment_sum(z2, edge_index)
  return _tc_decoder(agg, zr, W_nbr, W_out, b2.reshape(1, D))


# speedup vs baseline: 1.0014x; 1.0014x over previous
import functools

import jax
import jax.numpy as jnp
from jax import lax
from jax.experimental import pallas as pl
from jax.experimental.pallas import tpu as pltpu
from jax.experimental.pallas import tpu_sc as plsc

N = 10000
E = 320000
D = 128
DH = D // 2

NC = 2
NS = 16
EPT = E // NS
K = 128
NCHUNK = EPT // K
KTAIL = EPT - NCHUNK * K
NBUF = 4
ROWS_PER_TILE = 640
NPAD = ROWS_PER_TILE * NS
VL = 16


def _sc_segment_sum(z2, edge_index):
  mesh = plsc.VectorSubcoreMesh(core_axis_name="c", subcore_axis_name="s")

  @functools.partial(
      pl.kernel,
      mesh=mesh,
      compiler_params=pltpu.CompilerParams(use_tc_tiling_on_sc=False),
      out_type=jax.ShapeDtypeStruct((NPAD, D), jnp.float32),
      scratch_types=[
          pltpu.VMEM((EPT + 3 * K - EPT % K,), jnp.int32),
          pltpu.VMEM((EPT,), jnp.int32),
          [pltpu.VMEM((K, DH), jnp.float32)] * NBUF,
          pltpu.VMEM_SHARED((NPAD, DH), jnp.float32),
          [pltpu.SemaphoreType.DMA] * NBUF,
          [pltpu.SemaphoreType.DMA] * NBUF,
      ],
  )
  def k(z2_hbm, ei_hbm, out_hbm, src_v, dst_v, rows, acc, gsem, ssem):
    c = lax.axis_index("c")
    s = lax.axis_index("s")

    def zbody(i, carry):
      for u in range(DH // VL):
        rows[0][i, pl.ds(u * VL, VL)] = jnp.zeros((VL,), jnp.float32)
      return carry

    lax.fori_loop(0, K, zbody, 0)
    r0 = s * ROWS_PER_TILE
    for q in range(ROWS_PER_TILE // K):
      pltpu.sync_copy(rows[0],
                      acc.at[pl.ds(r0 + q * K, K)])
    pltpu.sync_copy(ei_hbm.at[0, pl.ds(s * EPT, EPT)],
                    src_v.at[pl.ds(0, EPT)])
    pltpu.sync_copy(ei_hbm.at[1, pl.ds(s * EPT, EPT)], dst_v)

    def transform(j):
      for u in range(K // VL):
        off = j * K + u * VL
        src_v[pl.ds(off, VL)] = src_v[pl.ds(off, VL)] * 2 + c

    transform(0)
    plsc.subcore_barrier()

    def gidx(j):
      return src_v.at[pl.ds(j * K, K)]

    def didx(j):
      return dst_v.at[pl.ds(j * K, K)]

    def wait_gather(j, v):
      pltpu.make_async_copy(z2_hbm.at[gidx(j)], rows[v], gsem[v]).wait()

    def start_scatter(j, v):
      pltpu.async_copy(rows[v], acc.at[didx(j)], ssem[v], add=True)

    def wait_scatter(j, v):
      pltpu.make_async_copy(rows[v], acc.at[didx(j)], ssem[v]).wait()

    def body(i, carry):
      j0 = NBUF * i
      for v in range(NBUF):
        j = j0 + v

        @pl.when(j >= NBUF)
        def _():
          wait_scatter(j - NBUF, v)

        pltpu.async_copy(z2_hbm.at[gidx(j)], rows[v], gsem[v])
        transform(j + 1)

        @pl.when(j >= NBUF - 1)
        def _():
          jl = j - (NBUF - 1)
          wait_gather(jl, (v + 1) % NBUF)
          start_scatter(jl, (v + 1) % NBUF)

      return carry

    lax.fori_loop(0, NCHUNK // NBUF, body, 0)
    wait_scatter(NCHUNK - NBUF, (NCHUNK - NBUF) % NBUF)
    for r in range(NCHUNK - NBUF + 1, NCHUNK):
      v = r % NBUF
      wait_gather(r, v)
      pltpu.sync_copy(rows[v], acc.at[didx(r)], add=True)

    toff = NCHUNK * K
    tsrc = src_v.at[pl.ds(toff, KTAIL)]
    tdst = dst_v.at[pl.ds(toff, KTAIL)]
    trows = rows[0].at[pl.ds(0, KTAIL)]
    pltpu.async_copy(z2_hbm.at[tsrc], trows, gsem[0]).wait()
    pltpu.sync_copy(trows, acc.at[tdst], add=True)

    plsc.subcore_barrier()

    pltpu.sync_copy(acc.at[pl.ds(r0, ROWS_PER_TILE)],
                    out_hbm.at[pl.ds(r0, ROWS_PER_TILE), pl.ds(c * DH, DH)])

  return k(z2, edge_index)


def _tc_zr_body(z_ref, wr_ref, b1_ref, o_ref):
  o_ref[...] = (jnp.dot(z_ref[...], wr_ref[...],
                        preferred_element_type=jnp.float32) + b1_ref[...])


def _tc_zr(z, W_root, b1):
  BN = 2000
  return pl.pallas_call(
      _tc_zr_body,
      grid=(N // BN,),
      in_specs=[
          pl.BlockSpec((BN, D), lambda i: (i, 0)),
          pl.BlockSpec((D, D), lambda i: (0, 0)),
          pl.BlockSpec((1, D), lambda i: (0, 0)),
      ],
      out_specs=pl.BlockSpec((BN, D), lambda i: (i, 0)),
      out_shape=jax.ShapeDtypeStruct((N, D), jnp.float32),
  )(z, W_root, b1)


def _tc_body(agg_ref, zr_ref, wn_ref, wo_ref, b2_ref, o_ref):
  h = zr_ref[...] + jnp.dot(agg_ref[...], wn_ref[...],
                            preferred_element_type=jnp.float32)
  h = jnp.maximum(h, 0.0)
  o_ref[...] = jnp.dot(h, wo_ref[...], preferred_element_type=jnp.float32) + b2_ref[...]


def _tc_decoder(agg, zr, W_nbr, W_out, b2):
  BN = 2000
  return pl.pallas_call(
      _tc_body,
      grid=(N // BN,),
      in_specs=[
          pl.BlockSpec((BN, D), lambda i: (i, 0)),
          pl.BlockSpec((BN, D), lambda i: (i, 0)),
          pl.BlockSpec((D, D), lambda i: (0, 0)),
          pl.BlockSpec((D, D), lambda i: (0, 0)),
          pl.BlockSpec((1, D), lambda i: (0, 0)),
      ],
      out_specs=pl.BlockSpec((BN, D), lambda i: (i, 0)),
      out_shape=jax.ShapeDtypeStruct((N, D), jnp.float32),
  )(agg, zr, W_nbr, W_out, b2)


def kernel(z, edge_index, W_root, W_nbr, b1, W_out, b2):
  z2 = z.reshape(2 * N, DH)
  zr = _tc_zr(z, W_root, b1.reshape(1, D))
  agg = _sc_segment_sum(z2, edge_index)
  return _tc_decoder(agg, zr, W_nbr, W_out, b2.reshape(1, D))
